# Initial kernel scaffold; baseline (speedup 1.0000x reference)
import functools

import jax
import jax.numpy as jnp
from jax import lax
from jax.experimental import pallas as pl
from jax.experimental.pallas import tpu as pltpu
from jax.experimental.pallas import tpu_sc as plsc

N, E, DIN, H, D, DOUT = 10000, 320000, 128, 8, 16, 16
HD = H * D

NC, NS, L = 2, 16, 16
C = 128
N_PAD = 10240
ROWS_PER_TILE = N_PAD // NS
E_PAD = ((E + NS * C - 1) // (NS * C)) * (NS * C)
EPT = E_PAD // NS
CHUNKS = EPT // C


_BM1 = 256


def _k1_body(x_ref, w_ref, a_ref, feat_ref, eler_ref):
    f = jnp.dot(x_ref[...], w_ref[0], preferred_element_type=jnp.float32)
    feat_ref[...] = f
    eler_ref[...] = jnp.dot(f, a_ref[0], preferred_element_type=jnp.float32)


def _run_k1(x_cat, w_stack, a_stack):
    grid = (2 * N_PAD) // _BM1
    per_mp = N_PAD // _BM1
    return pl.pallas_call(
        _k1_body,
        grid=(grid,),
        in_specs=[
            pl.BlockSpec((_BM1, DIN), lambda i: (i, 0)),
            pl.BlockSpec((1, DIN, HD), lambda i: (i // per_mp, 0, 0)),
            pl.BlockSpec((1, HD, HD), lambda i: (i // per_mp, 0, 0)),
        ],
        out_specs=[
            pl.BlockSpec((_BM1, HD), lambda i: (i, 0)),
            pl.BlockSpec((_BM1, HD), lambda i: (i, 0)),
        ],
        out_shape=[
            jax.ShapeDtypeStruct((2 * N_PAD, HD), jnp.float32),
            jax.ShapeDtypeStruct((2 * N_PAD, HD), jnp.float32),
        ],
    )(x_cat, w_stack, a_stack)


def _sc_body(src_hbm, dst_hbm, feat_hbm, el_hbm, er_hbm, z_hbm,
             src_v, dst_v, dstb_v, el_v, er_v, feat_v, buf_v, den_v,
             acc_sh, den_sh, sem1, sem2, sem3):
    c = lax.axis_index("c")
    s = lax.axis_index("s")
    zero = jnp.zeros((L,), jnp.float32)

    def zrow(r, carry):
        for j in range(HD // L):
            buf_v[r, pl.ds(j * L, L)] = zero
        den_v[r, pl.ds(0, L)] = zero
        return carry

    lax.fori_loop(0, C, zrow, 0)
    for k in range(ROWS_PER_TILE // C):
        row0 = s * ROWS_PER_TILE + k * C
        pltpu.sync_copy(buf_v, acc_sh.at[pl.ds(row0, C), :])
        pltpu.sync_copy(den_v, den_sh.at[pl.ds(row0, C), :])
    plsc.subcore_barrier()

    ebase = c * E_PAD + s * EPT
    bias = c * N_PAD

    def chunk_body(i, carry):
        base = ebase + i * C
        pltpu.sync_copy(src_hbm.at[pl.ds(base, C)], src_v)
        pltpu.sync_copy(dst_hbm.at[pl.ds(base, C)], dst_v)
        for j in range(C // L):
            dstb_v[pl.ds(j * L, L)] = dst_v[pl.ds(j * L, L)] + bias
        cp1 = pltpu.async_copy(el_hbm.at[src_v], el_v, sem1)
        cp2 = pltpu.async_copy(er_hbm.at[dstb_v], er_v, sem2)
        cp3 = pltpu.async_copy(feat_hbm.at[src_v], feat_v, sem3)
        cp1.wait()
        cp2.wait()

        def exrow(r, carry2):
            v = el_v[r, :] + er_v[r, :]
            v = jnp.where(v > 0.0, v, 0.2 * v)
            el_v[r, :] = jnp.exp(v)
            return carry2

        lax.fori_loop(0, C, exrow, 0)
        cp3.wait()

        def scalerow(r, carry2):
            for h in range(H):
                sval = el_v[r, h]
                feat_v[r, pl.ds(h * D, D)] = feat_v[r, pl.ds(h * D, D)] * sval
            return carry2

        lax.fori_loop(0, C, scalerow, 0)

        pltpu.sync_copy(feat_v, acc_sh.at[dst_v], add=True)
        pltpu.sync_copy(el_v, den_sh.at[dst_v], add=True)
        return carry

    lax.fori_loop(0, CHUNKS, chunk_body, 0)
    plsc.subcore_barrier()

    for k in range(ROWS_PER_TILE // C):
        row0 = s * ROWS_PER_TILE + k * C
        pltpu.sync_copy(acc_sh.at[pl.ds(row0, C), :], buf_v)
        pltpu.sync_copy(den_sh.at[pl.ds(row0, C), :], den_v)

        def normrow(r, carry):
            for h in range(H):
                dval = den_v[r, h] + 1e-9
                qv = buf_v[r, pl.ds(h * D, D)] / dval
                buf_v[r, pl.ds(h * D, D)] = jnp.where(
                    qv > 0.0, qv, jnp.exp(qv) - 1.0)
            return carry

        lax.fori_loop(0, C, normrow, 0)
        pltpu.sync_copy(buf_v, z_hbm.at[pl.ds(bias + row0, C), :])


@functools.partial(
    pl.kernel,
    out_type=jax.ShapeDtypeStruct((2 * N_PAD, HD), jnp.float32),
    mesh=plsc.VectorSubcoreMesh(core_axis_name="c", subcore_axis_name="s"),
    scratch_types=[
        pltpu.VMEM((C,), jnp.int32),
        pltpu.VMEM((C,), jnp.int32),
        pltpu.VMEM((C,), jnp.int32),
        pltpu.VMEM((C, L), jnp.float32),
        pltpu.VMEM((C, L), jnp.float32),
        pltpu.VMEM((C, HD), jnp.float32),
        pltpu.VMEM((C, HD), jnp.float32),
        pltpu.VMEM((C, L), jnp.float32),
        pltpu.VMEM_SHARED((N_PAD, HD), jnp.float32),
        pltpu.VMEM_SHARED((N_PAD, L), jnp.float32),
        pltpu.SemaphoreType.DMA,
        pltpu.SemaphoreType.DMA,
        pltpu.SemaphoreType.DMA,
    ],
)
def _sc_edge(src_hbm, dst_hbm, feat_hbm, el_hbm, er_hbm, z_hbm, *rest):
    _sc_body(src_hbm, dst_hbm, feat_hbm, el_hbm, er_hbm, z_hbm, *rest)


_BM2 = 200
_G2 = N // _BM2


def _k2a_body(z0_ref, z1_ref, ws_ref, bs_ref, q_ref, o0_ref, o1_ref):
    p0 = jnp.tanh(jnp.dot(z0_ref[...], ws_ref[...],
                          preferred_element_type=jnp.float32) + bs_ref[...])
    p1 = jnp.tanh(jnp.dot(z1_ref[...], ws_ref[...],
                          preferred_element_type=jnp.float32) + bs_ref[...])
    o0_ref[...] = jnp.full((1, HD), jnp.sum(p0 * q_ref[...]), jnp.float32)
    o1_ref[...] = jnp.full((1, HD), jnp.sum(p1 * q_ref[...]), jnp.float32)


def _run_k2a(z0, z1, Ws, bs2, q2):
    return pl.pallas_call(
        _k2a_body,
        grid=(_G2,),
        in_specs=[
            pl.BlockSpec((_BM2, HD), lambda i: (i, 0)),
            pl.BlockSpec((_BM2, HD), lambda i: (i, 0)),
            pl.BlockSpec((HD, HD), lambda i: (0, 0)),
            pl.BlockSpec((1, HD), lambda i: (0, 0)),
            pl.BlockSpec((1, HD), lambda i: (0, 0)),
        ],
        out_specs=[
            pl.BlockSpec((1, HD), lambda i: (i, 0)),
            pl.BlockSpec((1, HD), lambda i: (i, 0)),
        ],
        out_shape=[
            jax.ShapeDtypeStruct((_G2, HD), jnp.float32),
            jax.ShapeDtypeStruct((_G2, HD), jnp.float32),
        ],
    )(z0, z1, Ws, bs2, q2)


def _k2b_body(z0_ref, z1_ref, o0_ref, o1_ref, wl_ref, bl_ref, out_ref):
    w0 = jnp.sum(o0_ref[...]) / (HD * N)
    w1 = jnp.sum(o1_ref[...]) / (HD * N)
    m = jnp.maximum(w0, w1)
    b0 = jnp.exp(w0 - m)
    b1 = jnp.exp(w1 - m)
    tot = b0 + b1
    fused = (b0 / tot) * z0_ref[...] + (b1 / tot) * z1_ref[...]
    out_ref[...] = jnp.dot(fused, wl_ref[...],
                           preferred_element_type=jnp.float32) + bl_ref[...]


def _run_k2b(z0, z1, o0, o1, wl_pad, bl_pad):
    return pl.pallas_call(
        _k2b_body,
        grid=(_G2,),
        in_specs=[
            pl.BlockSpec((_BM2, HD), lambda i: (i, 0)),
            pl.BlockSpec((_BM2, HD), lambda i: (i, 0)),
            pl.BlockSpec((_G2, HD), lambda i: (0, 0)),
            pl.BlockSpec((_G2, HD), lambda i: (0, 0)),
            pl.BlockSpec((HD, HD), lambda i: (0, 0)),
            pl.BlockSpec((1, HD), lambda i: (0, 0)),
        ],
        out_specs=pl.BlockSpec((_BM2, HD), lambda i: (i, 0)),
        out_shape=jax.ShapeDtypeStruct((N, HD), jnp.float32),
    )(z0, z1, o0, o1, wl_pad, bl_pad)


def kernel(x, edge_index_mp0, edge_index_mp1, W0, al0, ar0, W1, al1, ar1,
           Ws, bs, q, Wlin, blin):
    f32 = jnp.float32
    x_pad = jnp.pad(x, ((0, N_PAD - N), (0, 0)))
    x_cat = jnp.concatenate([x_pad, x_pad], axis=0)
    w_stack = jnp.stack([W0, W1])

    onehot = jnp.repeat(jnp.eye(H, dtype=f32), D, axis=0)

    def mk_a(al, ar):
        a = jnp.zeros((HD, HD), f32)
        a = a.at[:, 0:H].set(onehot * al.reshape(-1, 1))
        a = a.at[:, L:L + H].set(onehot * ar.reshape(-1, 1))
        return a

    a_stack = jnp.stack([mk_a(al0, ar0), mk_a(al1, ar1)])

    feat_cat, eler_cat = _run_k1(x_cat, w_stack, a_stack)
    el_cat = eler_cat[:, 0:L]
    er_cat = eler_cat[:, L:2 * L]

    padw = E_PAD - E
    src_cat = jnp.concatenate([
        jnp.pad(edge_index_mp0[0], (0, padw), constant_values=N),
        jnp.pad(edge_index_mp1[0], (0, padw), constant_values=N) + N_PAD,
    ])
    dst_cat = jnp.concatenate([
        jnp.pad(edge_index_mp0[1], (0, padw), constant_values=N),
        jnp.pad(edge_index_mp1[1], (0, padw), constant_values=N),
    ])

    z_cat = _sc_edge(src_cat, dst_cat, feat_cat, el_cat, er_cat)
    z0 = z_cat[0:N]
    z1 = z_cat[N_PAD:N_PAD + N]

    bs2 = bs.reshape(1, HD)
    q2 = q.reshape(1, HD)
    o0, o1 = _run_k2a(z0, z1, Ws, bs2, q2)

    wl_pad = jnp.zeros((HD, HD), f32).at[:, 0:DOUT].set(Wlin)
    bl_pad = jnp.zeros((1, HD), f32).at[0, 0:DOUT].set(blin)
    out = _run_k2b(z0, z1, o0, o1, wl_pad, bl_pad)
    return out[:, 0:DOUT]


# speedup vs baseline: 9.5361x; 9.5361x over previous
import functools

import jax
import jax.numpy as jnp
from jax import lax
from jax.experimental import pallas as pl
from jax.experimental.pallas import tpu as pltpu
from jax.experimental.pallas import tpu_sc as plsc

N, E, DIN, H, D, DOUT = 10000, 320000, 128, 8, 16, 16
HD = H * D

NC, NS, L = 2, 16, 16
C = 128
N_PAD = 10240
QTR = N_PAD // 4
TRASH = 128
ACC_ROWS = QTR + TRASH
FB = 64
ACC_BLKS = ACC_ROWS // FB
QTR_BLKS = QTR // FB
CE = 64
E_PAD = ((E + NS * C - 1) // (NS * C)) * (NS * C)
EPT = E_PAD // NS
CHUNKS = EPT // CE


_BM1 = 256


def _k1_body(x_ref, w_ref, a_ref, fx_ref):
    f = jnp.dot(x_ref[...], w_ref[0], preferred_element_type=jnp.float32)
    fx_ref[:, 0, :] = f
    fx_ref[:, 1, :] = jnp.dot(f, a_ref[0], preferred_element_type=jnp.float32)


def _run_k1(x_cat, w_stack, a_stack):
    grid = (2 * N_PAD) // _BM1
    per_mp = N_PAD // _BM1
    return pl.pallas_call(
        _k1_body,
        grid=(grid,),
        in_specs=[
            pl.BlockSpec((_BM1, DIN), lambda i: (i, 0)),
            pl.BlockSpec((1, DIN, HD), lambda i: (i // per_mp, 0, 0)),
            pl.BlockSpec((1, HD, HD), lambda i: (i // per_mp, 0, 0)),
        ],
        out_specs=pl.BlockSpec((_BM1, 2, HD), lambda i: (i, 0, 0)),
        out_shape=jax.ShapeDtypeStruct((2 * N_PAD, 2, HD), jnp.float32),
    )(x_cat, w_stack, a_stack)


def _sc_body(src2_hbm, dst_hbm, featy0_hbm, featy1_hbm, er16_hbm, z_hbm,
             src2_v, dst_v, dstl_v, erow_v, el_v, fy_v, msg_v, dn_v, er_tile,
             acc_sh, den_sh):
    c = lax.axis_index("c")
    s = lax.axis_index("s")
    zero = jnp.zeros((L,), jnp.float32)
    fy64 = fy_v.at[pl.ds(0, FB), :]

    def zrow(r, carry):
        for j in range(HD // L):
            fy_v[r, pl.ds(j * L, L)] = zero
        return carry

    def dnrow(r, carry):
        for j in range(HD // L):
            dn_v[r, pl.ds(j * L, L)] = zero
        return carry

    def exzrow(r, carry):
        for j in range(HD // L):
            el_v[r, pl.ds(j * L, L)] = zero
        return carry

    def zero_acc():
        for k in range(ACC_BLKS // NS + 1):
            blk = s + k * NS

            @pl.when(blk < ACC_BLKS)
            def _():
                pltpu.sync_copy(fy64, acc_sh.at[pl.ds(blk * FB, FB), :])
                pltpu.sync_copy(dn_v, den_sh.at[pl.ds(blk * FB, FB), :])

    lax.fori_loop(0, C, zrow, 0)
    lax.fori_loop(0, FB, dnrow, 0)
    lax.fori_loop(0, CE, exzrow, 0)
    zero_acc()
    plsc.subcore_barrier()

    phases = [(p, q) for p in range(2) for q in range(2)]
    for pi, (p, q) in enumerate(phases):
        ebase = p * E_PAD + s * EPT
        featy_hbm = featy0_hbm if p == 0 else featy1_hbm
        lo = (2 * q + c) * QTR

        for cc in range(NC):
            @pl.when(c == cc)
            def _():
                er_row0 = p * (N_PAD // 8) + (2 * q + cc) * (QTR // 8)
                pltpu.sync_copy(
                    er16_hbm.at[pl.ds(er_row0, QTR // 8), :], er_tile)

        def chunk_body(i, carry):
            base = ebase + i * CE
            pltpu.sync_copy(src2_hbm.at[pl.ds(2 * base, C)], src2_v)
            pltpu.sync_copy(dst_hbm.at[pl.ds(base, CE)], dst_v)
            for j in range(CE // L):
                dv = dst_v[pl.ds(j * L, L)]
                off = dv - lo
                owned = (off >= 0) & (off < QTR)
                dstl_v[pl.ds(j * L, L)] = jnp.where(
                    owned, off, QTR + (dv & (TRASH - 1)))
                erow_v[pl.ds(j * L, L)] = jnp.clip(off, 0, QTR - 1)
            pltpu.sync_copy(featy_hbm.at[src2_v], fy_v)

            def exgroup(g, carry2):
                dl16 = erow_v[pl.ds(g * L, L)]
                for k in range(L):
                    r = g * L + k
                    dl = dl16[k]
                    erv = er_tile[dl >> 3, pl.ds((dl & 7) * L, L)]
                    v = fy_v[2 * r + 1, pl.ds(0, L)] + erv
                    v = jnp.where(v > 0.0, v, 0.2 * v)
                    el_v[r, pl.ds(0, L)] = jnp.exp(v)
                return carry2

            lax.fori_loop(0, CE // L, exgroup, 0)

            def scalerow(r, carry2):
                exv = el_v[r, pl.ds(0, L)]
                for h in range(H):
                    sval = exv[h]
                    msg_v[r, pl.ds(h * D, D)] = (
                        fy_v[2 * r, pl.ds(h * D, D)] * sval)
                return carry2

            lax.fori_loop(0, CE, scalerow, 0)

            pltpu.sync_copy(msg_v, acc_sh.at[dstl_v], add=True)
            pltpu.sync_copy(el_v, den_sh.at[dstl_v], add=True)
            return carry

        lax.fori_loop(0, CHUNKS, chunk_body, 0)
        plsc.subcore_barrier()

        for k in range(QTR_BLKS // NS + 1):
            blk = s + k * NS

            @pl.when(blk < QTR_BLKS)
            def _():
                row0 = blk * FB
                pltpu.sync_copy(acc_sh.at[pl.ds(row0, FB), :], fy64)
                pltpu.sync_copy(den_sh.at[pl.ds(row0, FB), :], dn_v)

                def normrow(r, carry):
                    denv = dn_v[r, pl.ds(0, L)]
                    for h in range(H):
                        dval = denv[h] + 1e-9
                        qv = fy_v[r, pl.ds(h * D, D)] / dval
                        fy_v[r, pl.ds(h * D, D)] = jnp.where(
                            qv > 0.0, qv, jnp.exp(qv) - 1.0)
                    return carry

                lax.fori_loop(0, FB, normrow, 0)
                zrow0 = pl.multiple_of(p * N_PAD + lo + row0, 8)
                pltpu.sync_copy(fy64, z_hbm.at[pl.ds(zrow0, FB), :])

        if pi < len(phases) - 1:
            plsc.subcore_barrier()
            lax.fori_loop(0, C, zrow, 0)
            lax.fori_loop(0, FB, dnrow, 0)
            zero_acc()
            plsc.subcore_barrier()


@functools.partial(
    pl.kernel,
    out_type=jax.ShapeDtypeStruct((2 * N_PAD, HD), jnp.float32),
    mesh=plsc.VectorSubcoreMesh(core_axis_name="c", subcore_axis_name="s"),
    scratch_types=[
        pltpu.VMEM((C,), jnp.int32),
        pltpu.VMEM((CE,), jnp.int32),
        pltpu.VMEM((CE,), jnp.int32),
        pltpu.VMEM((CE,), jnp.int32),
        pltpu.VMEM((CE, HD), jnp.float32),
        pltpu.VMEM((C, HD), jnp.float32),
        pltpu.VMEM((CE, HD), jnp.float32),
        pltpu.VMEM((FB, HD), jnp.float32),
        pltpu.VMEM((QTR // 8, HD), jnp.float32),
        pltpu.VMEM_SHARED((ACC_ROWS, HD), jnp.float32),
        pltpu.VMEM_SHARED((ACC_ROWS, HD), jnp.float32),
    ],
)
def _sc_edge(src2_hbm, dst_hbm, featy0_hbm, featy1_hbm, er16_hbm, z_hbm,
             *rest):
    _sc_body(src2_hbm, dst_hbm, featy0_hbm, featy1_hbm, er16_hbm, z_hbm,
             *rest)


_BM2 = 200
_G2 = N // _BM2


def _k2a_body(z0_ref, z1_ref, ws_ref, bs_ref, q_ref, o0_ref, o1_ref):
    p0 = jnp.tanh(jnp.dot(z0_ref[...], ws_ref[...],
                          preferred_element_type=jnp.float32) + bs_ref[...])
    p1 = jnp.tanh(jnp.dot(z1_ref[...], ws_ref[...],
                          preferred_element_type=jnp.float32) + bs_ref[...])
    o0_ref[...] = jnp.full((8, HD), jnp.sum(p0 * q_ref[...]), jnp.float32)
    o1_ref[...] = jnp.full((8, HD), jnp.sum(p1 * q_ref[...]), jnp.float32)


def _run_k2a(z0, z1, Ws, bs2, q2):
    return pl.pallas_call(
        _k2a_body,
        grid=(_G2,),
        in_specs=[
            pl.BlockSpec((_BM2, HD), lambda i: (i, 0)),
            pl.BlockSpec((_BM2, HD), lambda i: (i, 0)),
            pl.BlockSpec((HD, HD), lambda i: (0, 0)),
            pl.BlockSpec((1, HD), lambda i: (0, 0)),
            pl.BlockSpec((1, HD), lambda i: (0, 0)),
        ],
        out_specs=[
            pl.BlockSpec((8, HD), lambda i: (i, 0)),
            pl.BlockSpec((8, HD), lambda i: (i, 0)),
        ],
        out_shape=[
            jax.ShapeDtypeStruct((_G2 * 8, HD), jnp.float32),
            jax.ShapeDtypeStruct((_G2 * 8, HD), jnp.float32),
        ],
    )(z0, z1, Ws, bs2, q2)


def _k2b_body(z0_ref, z1_ref, o0_ref, o1_ref, wl_ref, bl_ref, out_ref):
    w0 = jnp.sum(o0_ref[...]) / (8 * HD * N)
    w1 = jnp.sum(o1_ref[...]) / (8 * HD * N)
    m = jnp.maximum(w0, w1)
    b0 = jnp.exp(w0 - m)
    b1 = jnp.exp(w1 - m)
    tot = b0 + b1
    fused = (b0 / tot) * z0_ref[...] + (b1 / tot) * z1_ref[...]
    out_ref[...] = jnp.dot(fused, wl_ref[...],
                           preferred_element_type=jnp.float32) + bl_ref[...]


def _run_k2b(z0, z1, o0, o1, wl_pad, bl_pad):
    return pl.pallas_call(
        _k2b_body,
        grid=(_G2,),
        in_specs=[
            pl.BlockSpec((_BM2, HD), lambda i: (i, 0)),
            pl.BlockSpec((_BM2, HD), lambda i: (i, 0)),
            pl.BlockSpec((_G2 * 8, HD), lambda i: (0, 0)),
            pl.BlockSpec((_G2 * 8, HD), lambda i: (0, 0)),
            pl.BlockSpec((HD, HD), lambda i: (0, 0)),
            pl.BlockSpec((1, HD), lambda i: (0, 0)),
        ],
        out_specs=pl.BlockSpec((_BM2, HD), lambda i: (i, 0)),
        out_shape=jax.ShapeDtypeStruct((N, HD), jnp.float32),
    )(z0, z1, o0, o1, wl_pad, bl_pad)


def kernel(x, edge_index_mp0, edge_index_mp1, W0, al0, ar0, W1, al1, ar1,
           Ws, bs, q, Wlin, blin):
    f32 = jnp.float32
    x_pad = jnp.pad(x, ((0, N_PAD - N), (0, 0)))
    x_cat = jnp.concatenate([x_pad, x_pad], axis=0)
    w_stack = jnp.stack([W0, W1])

    onehot = jnp.repeat(jnp.eye(H, dtype=f32), D, axis=0)

    def mk_a(al, ar):
        a = jnp.zeros((HD, HD), f32)
        a = a.at[:, 0:H].set(onehot * al.reshape(-1, 1))
        a = a.at[:, L:L + H].set(onehot * ar.reshape(-1, 1))
        return a

    a_stack = jnp.stack([mk_a(al0, ar0), mk_a(al1, ar1)])

    featx3 = _run_k1(x_cat, w_stack, a_stack)
    featy0 = featx3[:N_PAD].reshape(2 * N_PAD, HD)
    featy1 = featx3[N_PAD:].reshape(2 * N_PAD, HD)
    er16_cat = featx3[:, 1, L:2 * L]
    er128_cat = er16_cat.reshape(2 * N_PAD // 8, 8 * L)

    padw = E_PAD - E
    src_cat = jnp.concatenate([
        jnp.pad(edge_index_mp0[0], (0, padw), constant_values=N),
        jnp.pad(edge_index_mp1[0], (0, padw), constant_values=N),
    ])
    dst_cat = jnp.concatenate([
        jnp.pad(edge_index_mp0[1], (0, padw), constant_values=N),
        jnp.pad(edge_index_mp1[1], (0, padw), constant_values=N),
    ])
    src2_cat = jnp.stack([2 * src_cat, 2 * src_cat + 1], axis=1).reshape(-1)

    z_cat = _sc_edge(src2_cat, dst_cat, featy0, featy1, er128_cat)
    z0 = z_cat[0:N]
    z1 = z_cat[N_PAD:N_PAD + N]

    bs2 = bs.reshape(1, HD)
    q2 = q.reshape(1, HD)
    o0, o1 = _run_k2a(z0, z1, Ws, bs2, q2)

    wl_pad = jnp.zeros((HD, HD), f32).at[:, 0:DOUT].set(Wlin)
    bl_pad = jnp.zeros((1, HD), f32).at[0, 0:DOUT].set(blin)
    out = _run_k2b(z0, z1, o0, o1, wl_pad, bl_pad)
    return out[:, 0:DOUT]
